# baseline (speedup 1.0000x reference)
import functools

import jax
import jax.numpy as jnp
from jax import lax
from jax.experimental import pallas as pl
from jax.experimental.pallas import tpu as pltpu
from jax.experimental.pallas import tpu_sc as plsc

H = 1024
W = 1024
LANES = 16
SUB = 8
SUBW = W // SUB
GRP = SUBW // LANES


def _bf16_round(v):
    u = lax.bitcast_convert_type(v, jnp.uint32)
    up = u + jnp.uint32(0x7FFF) + ((u >> jnp.uint32(16)) & jnp.uint32(1))
    up = up & jnp.uint32(0xFFFF0000)
    return lax.bitcast_convert_type(up, jnp.float32)


def _make_kernel():
    info = plsc.get_sparse_core_info()
    nc, ns = info.num_cores, info.num_subcores
    nw = nc * ns
    rows_per_w = H // nw

    mesh = plsc.VectorSubcoreMesh(core_axis_name="c", subcore_axis_name="s")

    @functools.partial(
        pl.kernel,
        mesh=mesh,
        out_type=jax.ShapeDtypeStruct((H, SUB, SUBW), jnp.float32),
        scratch_types=[
            pltpu.VMEM((6, LANES), jnp.float32),
            pltpu.VMEM((2, SUB, SUBW), jnp.float32),
            pltpu.VMEM((2, 4 * SUB, SUBW), jnp.int32),
            pltpu.VMEM((2, 4 * SUB, SUBW), jnp.float32),
            pltpu.VMEM((2, SUB, SUBW), jnp.float32),
            pltpu.VMEM((2, SUB, SUBW), jnp.float32),
            pltpu.VMEM((2, SUB, SUBW), jnp.float32),
            pltpu.VMEM((SUB, SUBW), jnp.float32),
            pltpu.SemaphoreType.DMA,
            pltpu.SemaphoreType.DMA,
        ],
    )
    def spatial_tx(img_hbm, th_hbm, out_hbm, th_v, xc_v, idx_v, cor_v, fx_v,
                   fy_v, va_v, orow_v, sem0, sem1):
        sems = (sem0, sem1)
        wid = lax.axis_index("s") * nc + lax.axis_index("c")
        pltpu.sync_copy(th_hbm, th_v)
        a00 = _bf16_round(th_v[0])
        a01 = _bf16_round(th_v[1])
        a02 = _bf16_round(th_v[2])
        a10 = _bf16_round(th_v[3])
        a11 = _bf16_round(th_v[4])
        a12 = _bf16_round(th_v[5])
        base_row = wid * rows_per_w
        lane = lax.iota(jnp.int32, LANES)

        def col_body(j, c1):
            def grp_body(k, c2):
                w0 = j * SUBW + k * LANES
                wv = jnp.full((LANES,), w0, dtype=jnp.int32) + lane
                xn = _bf16_round(wv.astype(jnp.float32) / 1023.0)
                sl = pl.ds(k * LANES, LANES)
                xc_v[0, j, sl] = a00 * xn
                xc_v[1, j, sl] = a10 * xn
                return c2

            return lax.fori_loop(0, GRP, grp_body, c1)

        lax.fori_loop(0, SUB, col_body, 0)

        def compute_idx(h, slot):
            yn = _bf16_round(
                jnp.full((LANES,), h, dtype=jnp.int32).astype(jnp.float32)
                / 1023.0)
            y0s = a01 * yn
            y1s = a11 * yn

            def idx_body(j, c1):
                def grp_body(k, c2):
                    sl = pl.ds(k * LANES, LANES)
                    xs = (xc_v[0, j, sl] + y0s + a02) * 1023.0
                    ys = (xc_v[1, j, sl] + y1s + a12) * 1023.0
                    x0 = jnp.clip(xs.astype(jnp.int32), 0, W - 2)
                    y0 = jnp.clip(ys.astype(jnp.int32), 0, H - 2)
                    fx = xs - x0.astype(jnp.float32)
                    fy = ys - y0.astype(jnp.float32)
                    ok = ((xs >= 0.0) & (xs < 1023.0)
                          & (ys >= 0.0) & (ys < 1023.0))
                    vf = jnp.where(ok, jnp.float32(1.0), jnp.float32(0.0))
                    b = y0 * W + x0
                    idx_v[slot, j, sl] = b
                    idx_v[slot, SUB + j, sl] = b + 1
                    idx_v[slot, 2 * SUB + j, sl] = b + W
                    idx_v[slot, 3 * SUB + j, sl] = b + W + 1
                    fx_v[slot, j, sl] = fx
                    fy_v[slot, j, sl] = fy
                    va_v[slot, j, sl] = vf
                    return c2

                return lax.fori_loop(0, GRP, grp_body, c1)

            lax.fori_loop(0, SUB, idx_body, 0)

        def issue(slot):
            for i in range(4 * SUB):
                pltpu.async_copy(img_hbm.at[idx_v.at[slot, i]],
                                 cor_v.at[slot, i], sems[slot])

        def drain(slot):
            for i in range(4 * SUB):
                pltpu.make_async_copy(img_hbm.at[idx_v.at[slot, i]],
                                      cor_v.at[slot, i], sems[slot]).wait()

        def blend_write(h, slot):
            def out_body(j, c1):
                def grp_body(k, c2):
                    sl = pl.ds(k * LANES, LANES)
                    ia = cor_v[slot, j, sl]
                    ic = cor_v[slot, SUB + j, sl]
                    ib = cor_v[slot, 2 * SUB + j, sl]
                    idd = cor_v[slot, 3 * SUB + j, sl]
                    fx = fx_v[slot, j, sl]
                    fy = fy_v[slot, j, sl]
                    vf = va_v[slot, j, sl]
                    gx = 1.0 - fx
                    gy = 1.0 - fy
                    top = gx * ia + fx * ic
                    bot = gx * ib + fx * idd
                    orow_v[j, sl] = vf * (gy * top + fy * bot)
                    return c2

                return lax.fori_loop(0, GRP, grp_body, c1)

            lax.fori_loop(0, SUB, out_body, 0)
            pltpu.sync_copy(orow_v, out_hbm.at[h])

        compute_idx(base_row, 0)
        issue(0)

        def ring_body(g, carry):
            r0 = base_row + 2 * g
            compute_idx(r0 + 1, 1)
            issue(1)
            drain(0)
            blend_write(r0, 0)
            compute_idx(r0 + 2, 0)
            issue(0)
            drain(1)
            blend_write(r0 + 1, 1)
            return carry

        lax.fori_loop(0, rows_per_w // 2, ring_body, 0)
        drain(0)

    return spatial_tx


_SPATIAL_TX = _make_kernel()


def kernel(input_fmap, theta, B):
    img = input_fmap.reshape(H * W)
    th = jnp.broadcast_to(theta.astype(jnp.float32).reshape(6, 1), (6, LANES))
    out = _SPATIAL_TX(img, th)
    return out.reshape(1, H, W, 1)


# speedup vs baseline: 9.3128x; 9.3128x over previous
import functools

import jax
import jax.numpy as jnp
from jax import lax
from jax.experimental import pallas as pl
from jax.experimental.pallas import tpu as pltpu
from jax.experimental.pallas import tpu_sc as plsc

H = 1024
W = 1024
LANES = 16
SUB = 8
SUBW = W // SUB
GRP = SUBW // LANES


def _bf16_round(v):
    u = lax.bitcast_convert_type(v, jnp.uint32)
    up = u + jnp.uint32(0x7FFF) + ((u >> jnp.uint32(16)) & jnp.uint32(1))
    up = up & jnp.uint32(0xFFFF0000)
    return lax.bitcast_convert_type(up, jnp.float32)


def _make_kernel():
    info = plsc.get_sparse_core_info()
    nc, ns = info.num_cores, info.num_subcores
    nw = nc * ns
    rows_per_w = H // nw

    mesh = plsc.VectorSubcoreMesh(core_axis_name="c", subcore_axis_name="s")

    @functools.partial(
        pl.kernel,
        mesh=mesh,
        out_type=jax.ShapeDtypeStruct((H, SUB, SUBW), jnp.float32),
        scratch_types=[
            pltpu.VMEM_SHARED((H * W,), jnp.float32),
            pltpu.VMEM((6, LANES), jnp.float32),
            pltpu.VMEM((2, SUB, SUBW), jnp.float32),
            pltpu.VMEM((2, 4 * SUB, SUBW), jnp.int32),
            pltpu.VMEM((2, 4 * SUB, SUBW), jnp.float32),
            pltpu.VMEM((2, SUB, SUBW), jnp.float32),
            pltpu.VMEM((2, SUB, SUBW), jnp.float32),
            pltpu.VMEM((2, SUB, SUBW), jnp.float32),
            pltpu.VMEM((SUB, SUBW), jnp.float32),
            pltpu.SemaphoreType.DMA,
            pltpu.SemaphoreType.DMA,
        ],
    )
    def spatial_tx(img_hbm, th_hbm, out_hbm, spm, th_v, xc_v, idx_v, cor_v,
                   fx_v, fy_v, va_v, orow_v, sem0, sem1):
        sems = (sem0, sem1)
        sid = lax.axis_index("s")
        wid = sid * nc + lax.axis_index("c")
        chunk = (H * W) // ns
        pltpu.sync_copy(img_hbm.at[pl.ds(sid * chunk, chunk)],
                        spm.at[pl.ds(sid * chunk, chunk)])
        plsc.subcore_barrier()
        pltpu.sync_copy(th_hbm, th_v)
        a00 = _bf16_round(th_v[0])
        a01 = _bf16_round(th_v[1])
        a02 = _bf16_round(th_v[2])
        a10 = _bf16_round(th_v[3])
        a11 = _bf16_round(th_v[4])
        a12 = _bf16_round(th_v[5])
        base_row = wid * rows_per_w
        lane = lax.iota(jnp.int32, LANES)

        def col_body(j, c1):
            def grp_body(k, c2):
                w0 = j * SUBW + k * LANES
                wv = jnp.full((LANES,), w0, dtype=jnp.int32) + lane
                xn = _bf16_round(wv.astype(jnp.float32) / 1023.0)
                sl = pl.ds(k * LANES, LANES)
                xc_v[0, j, sl] = a00 * xn
                xc_v[1, j, sl] = a10 * xn
                return c2

            return lax.fori_loop(0, GRP, grp_body, c1)

        lax.fori_loop(0, SUB, col_body, 0)

        def compute_idx(h, slot):
            yn = _bf16_round(
                jnp.full((LANES,), h, dtype=jnp.int32).astype(jnp.float32)
                / 1023.0)
            y0s = a01 * yn
            y1s = a11 * yn

            def idx_body(j, c1):
                def grp_body(k, c2):
                    sl = pl.ds(k * LANES, LANES)
                    xs = (xc_v[0, j, sl] + y0s + a02) * 1023.0
                    ys = (xc_v[1, j, sl] + y1s + a12) * 1023.0
                    x0 = jnp.clip(xs.astype(jnp.int32), 0, W - 2)
                    y0 = jnp.clip(ys.astype(jnp.int32), 0, H - 2)
                    fx = xs - x0.astype(jnp.float32)
                    fy = ys - y0.astype(jnp.float32)
                    ok = ((xs >= 0.0) & (xs < 1023.0)
                          & (ys >= 0.0) & (ys < 1023.0))
                    vf = jnp.where(ok, jnp.float32(1.0), jnp.float32(0.0))
                    b = y0 * W + x0
                    idx_v[slot, j, sl] = b
                    idx_v[slot, SUB + j, sl] = b + 1
                    idx_v[slot, 2 * SUB + j, sl] = b + W
                    idx_v[slot, 3 * SUB + j, sl] = b + W + 1
                    fx_v[slot, j, sl] = fx
                    fy_v[slot, j, sl] = fy
                    va_v[slot, j, sl] = vf
                    return c2

                return lax.fori_loop(0, GRP, grp_body, c1)

            lax.fori_loop(0, SUB, idx_body, 0)

        def issue(slot):
            for i in range(4 * SUB):
                pltpu.async_copy(spm.at[idx_v.at[slot, i]],
                                 cor_v.at[slot, i], sems[slot])

        def drain(slot):
            for i in range(4 * SUB):
                pltpu.make_async_copy(img_hbm.at[idx_v.at[slot, i]],
                                      cor_v.at[slot, i], sems[slot]).wait()

        def blend_write(h, slot):
            def out_body(j, c1):
                def grp_body(k, c2):
                    sl = pl.ds(k * LANES, LANES)
                    ia = cor_v[slot, j, sl]
                    ic = cor_v[slot, SUB + j, sl]
                    ib = cor_v[slot, 2 * SUB + j, sl]
                    idd = cor_v[slot, 3 * SUB + j, sl]
                    fx = fx_v[slot, j, sl]
                    fy = fy_v[slot, j, sl]
                    vf = va_v[slot, j, sl]
                    gx = 1.0 - fx
                    gy = 1.0 - fy
                    top = gx * ia + fx * ic
                    bot = gx * ib + fx * idd
                    orow_v[j, sl] = vf * (gy * top + fy * bot)
                    return c2

                return lax.fori_loop(0, GRP, grp_body, c1)

            lax.fori_loop(0, SUB, out_body, 0)
            pltpu.sync_copy(orow_v, out_hbm.at[h])

        compute_idx(base_row, 0)
        issue(0)

        def ring_body(g, carry):
            r0 = base_row + 2 * g
            compute_idx(r0 + 1, 1)
            issue(1)
            drain(0)
            blend_write(r0, 0)
            compute_idx(r0 + 2, 0)
            issue(0)
            drain(1)
            blend_write(r0 + 1, 1)
            return carry

        lax.fori_loop(0, rows_per_w // 2, ring_body, 0)
        drain(0)

    return spatial_tx


_SPATIAL_TX = _make_kernel()


def kernel(input_fmap, theta, B):
    img = input_fmap.reshape(H * W)
    th = jnp.broadcast_to(theta.astype(jnp.float32).reshape(6, 1), (6, LANES))
    out = _SPATIAL_TX(img, th)
    return out.reshape(1, H, W, 1)
